# baseline (speedup 1.0000x reference)
import functools

import jax
import jax.numpy as jnp
from jax import lax
from jax.experimental import pallas as pl
from jax.experimental.pallas import tpu as pltpu
from jax.experimental.pallas import tpu_sc as plsc

B, LK, LF, D = 16, 1024, 2048, 768
NC, NS = 2, 16
NW = NC * NS
ROWS = B * LF
LPW = LF // NW
C = 32
NCH = B * LPW // C
RPT = B * LPW
VPR = D // 16
VCAP = 16


def _body(table, ridx, mask, mt, pos, out, S_res, mt_v, idxr_f, idxm_a, mch_a,
          cidx_a, nv_a, v0, v1, o0, o1, sem, sem_pos, sem_r,
          sem_in0, sem_in1, sem_out0, sem_out1):
    wid = lax.axis_index("s") * NC + lax.axis_index("c")
    l0 = wid * LPW
    iota = lax.iota(jnp.int32, 16)
    drow_a = idxm_a

    dpos = pltpu.async_copy(pos.at[pl.ds(l0, LPW)], S_res, sem_pos)
    dmt = pltpu.async_copy(mt.at[pl.ds(0, D)], mt_v, sem_r)
    descs = [pltpu.async_copy(ridx.at[pl.ds(b * LF + l0, LPW)],
                              idxr_f.at[pl.ds(b * LPW, LPW)], sem)
             for b in range(B)]
    for d in descs:
        d.wait()
    for k in range(RPT // 16):
        b = k // (LPW // 16)
        i = idxr_f[pl.ds(k * 16, 16)]
        idxm_a[pl.ds(k * 16, 16)] = b * LK + jnp.minimum(i, LK - 1)
    mdescs = [pltpu.async_copy(mask.at[idxm_a.at[pl.ds(j * 128, 128)]],
                               mch_a.at[pl.ds(j * 128, 128)], sem)
              for j in range(RPT // 128)]
    for d in mdescs:
        d.wait()
    zeros = jnp.zeros((16,), jnp.int32)
    for k in range(RPT // 16):
        cidx_a[pl.ds(k * 16, 16)] = zeros
    dpos.wait()
    dmt.wait()

    def srow(r, carry):
        for j in range(VPR):
            s = pl.ds(j * 16, 16)
            S_res[r, s] = S_res[r, s] + mt_v[s]
        return carry

    lax.fori_loop(0, LPW, srow, 0)

    def compact(c, carry):
        b = c // 2
        cnt = jnp.int32(0)
        for k in range(2):
            off = c * C + k * 16
            i = idxr_f[pl.ds(off, 16)]
            mv = mch_a[pl.ds(off, 16)]
            vint = jnp.where((i < LK) & (mv == 1), 1, 0)
            gidx = b * LK + i
            for lane in range(16):
                okv = vint[lane]
                gval = gidx[lane]
                wal = c * C + (cnt & ~15)
                lt = cnt & 15

                @pl.when(okv == 1)
                def _():
                    vw = cidx_a[pl.ds(wal, 16)]
                    cidx_a[pl.ds(wal, 16)] = jnp.where(iota == lt, gval, vw)
                    vd = drow_a[pl.ds(wal, 16)]
                    drow_a[pl.ds(wal, 16)] = jnp.where(
                        iota == lt, k * 16 + lane, vd)

                cnt = cnt + okv
        wal2 = c & ~15
        lt2 = c & 15
        vn = nv_a[pl.ds(wal2, 16)]
        nv_a[pl.ds(wal2, 16)] = jnp.where(iota == lt2, cnt, vn)
        return carry

    lax.fori_loop(0, NCH, compact, 0)

    def nv_of(x):
        return nv_a[pl.ds(x, 16)][0]

    def out_base(x):
        return (x // 2) * LF + l0 + (x & 1) * C

    def gdesc(x, sz, vbuf, sem_in):
        return pltpu.make_async_copy(
            table.at[cidx_a.at[pl.ds(x * C, sz)]], vbuf.at[pl.ds(0, sz)],
            sem_in)

    def bucket_start(x, vbuf, sem_in):
        nv = nv_of(x)

        @pl.when((nv > 0) & (nv <= 8))
        def _():
            gdesc(x, 8, vbuf, sem_in).start()

        @pl.when((nv > 8) & (nv <= VCAP))
        def _():
            gdesc(x, VCAP, vbuf, sem_in).start()

    def bucket_wait(x, vbuf, sem_in):
        nv = nv_of(x)

        @pl.when((nv > 0) & (nv <= 8))
        def _():
            gdesc(x, 8, vbuf, sem_in).wait()

        @pl.when((nv > 8) & (nv <= VCAP))
        def _():
            gdesc(x, VCAP, vbuf, sem_in).wait()

    def wdesc(x, obuf, sem_out):
        return pltpu.make_async_copy(obuf, out.at[pl.ds(out_base(x), C)],
                                     sem_out)

    def base_copy(x, obuf):
        poff = (x & 1) * C

        def row(r, carry):
            for j in range(VPR):
                s = pl.ds(j * 16, 16)
                obuf[r, s] = S_res[poff + r, s]
            return carry

        lax.fori_loop(0, C, row, 0)

    def valid_fix(x, obuf, vbuf, s_lo, s_hi):
        poff = (x & 1) * C
        dbase = x * C

        def one(s, carry):
            dr = drow_a[pl.ds(dbase + s, 16)][0]
            vs = s - s_lo
            for j in range(VPR):
                sl = pl.ds(j * 16, 16)
                obuf[dr, sl] = vbuf[vs, sl] + (S_res[poff + dr, sl] - mt_v[sl])
            return carry

        lax.fori_loop(s_lo, s_hi, one, 0)

    bufs_v = (v0, v1)
    bufs_o = (o0, o1)
    sin = (sem_in0, sem_in1)
    sout = (sem_out0, sem_out1)

    def chunk_body(x, p, first_guard, last_guard):
        vbuf, obuf = bufs_v[p], bufs_o[p]

        @pl.when(first_guard)
        def _():
            wdesc(x - 2, obuf, sout[p]).wait()

        base_copy(x, obuf)

        @pl.when(last_guard)
        def _():
            bucket_start(x + 1, bufs_v[1 - p], sin[1 - p])

        nv = nv_of(x)
        bucket_wait(x, vbuf, sin[p])

        @pl.when((nv > 0) & (nv <= VCAP))
        def _():
            valid_fix(x, obuf, vbuf, 0, nv)

        @pl.when(nv > VCAP)
        def _():
            gdesc(x, VCAP, vbuf, sem_r).start()
            gdesc(x, VCAP, vbuf, sem_r).wait()
            valid_fix(x, obuf, vbuf, 0, VCAP)
            pltpu.async_copy(
                table.at[cidx_a.at[pl.ds(x * C + VCAP, VCAP)]],
                vbuf.at[pl.ds(0, VCAP)], sem_r).wait()
            valid_fix(x, obuf, vbuf, VCAP, nv)

        wdesc(x, obuf, sout[p]).start()

    bucket_start(0, v0, sem_in0)

    def step(c2, carry):
        x = c2 * 2
        chunk_body(x, 0, first_guard=c2 > 0, last_guard=c2 >= 0)
        chunk_body(x + 1, 1, first_guard=c2 > 0, last_guard=c2 < NCH // 2 - 1)
        return carry

    lax.fori_loop(0, NCH // 2, step, 0)
    wdesc(NCH - 2, o0, sem_out0).wait()
    wdesc(NCH - 1, o1, sem_out1).wait()


@functools.partial(
    pl.kernel,
    out_type=jax.ShapeDtypeStruct((ROWS, D), jnp.float32),
    mesh=plsc.VectorSubcoreMesh(core_axis_name="c", subcore_axis_name="s",
                                num_cores=NC, num_subcores=NS),
    scratch_types=[
        pltpu.VMEM((LPW, D), jnp.float32),
        pltpu.VMEM((D,), jnp.float32),
        pltpu.VMEM((RPT,), jnp.int32),
        pltpu.VMEM((RPT + 16,), jnp.int32),
        pltpu.VMEM((RPT,), jnp.int32),
        pltpu.VMEM((RPT + 16,), jnp.int32),
        pltpu.VMEM((NCH + 16,), jnp.int32),
        pltpu.VMEM((VCAP, D), jnp.float32),
        pltpu.VMEM((VCAP, D), jnp.float32),
        pltpu.VMEM((C, D), jnp.float32),
        pltpu.VMEM((C, D), jnp.float32),
        pltpu.SemaphoreType.DMA,
        pltpu.SemaphoreType.DMA,
        pltpu.SemaphoreType.DMA,
        pltpu.SemaphoreType.DMA,
        pltpu.SemaphoreType.DMA,
        pltpu.SemaphoreType.DMA,
        pltpu.SemaphoreType.DMA,
    ],
)
def _revert_sc(table, ridx, mask, mt, pos, out, S_res, mt_v, idxr_f, idxm_a,
               mch_a, cidx_a, nv_a, v0, v1, o0, o1, sem, sem_pos, sem_r,
               sem_in0, sem_in1, sem_out0, sem_out1):
    _body(table, ridx, mask, mt, pos, out, S_res, mt_v, idxr_f, idxm_a, mch_a,
          cidx_a, nv_a, v0, v1, o0, o1, sem, sem_pos, sem_r,
          sem_in0, sem_in1, sem_out0, sem_out1)


def kernel(val, remain_padding_mask, revert_idx, mask_token, pos_enc):
    val2d = val.reshape(B * LK, D)
    ridx = revert_idx.reshape(ROWS).astype(jnp.int32)
    mask = remain_padding_mask.reshape(B * LK).astype(jnp.int32)
    out = _revert_sc(val2d, ridx, mask, mask_token.astype(jnp.float32),
                     pos_enc.astype(jnp.float32))
    return out.reshape(B, LF, D)
